# baseline (speedup 1.0000x reference)
import functools

import jax
import jax.numpy as jnp
from jax.experimental import pallas as pl
from jax.experimental.pallas import tpu as pltpu


def _shifted2(x, off, length):
    c, _ = x.shape
    if off < 0:
        zc = jnp.zeros((c, -off), jnp.float32)
        return jnp.concatenate([zc, x[:, : length + off]], axis=1)
    zc = jnp.zeros((c, off), jnp.float32)
    return jnp.concatenate([x[:, off:], zc], axis=1)


def _phased_kernel(x_ref, dw_ref, pw_ref, o_ref, z_scr, st_scr, sc_scr, *,
                   ksize, b_tile, count, eps):
    p = pl.program_id(0)
    b = pl.program_id(1)

    @pl.when(p == 0)
    def _conv():
        @pl.when(b == 0)
        def _init():
            st_scr[...] = jnp.zeros_like(st_scr)

        dw = dw_ref[...]
        pw = pw_ref[...]
        c_in, length = x_ref.shape[1], x_ref.shape[2]
        pad = (ksize - 1) // 2
        cc = 64
        ds = [dw[:, k].reshape(c_in, 1) for k in range(ksize)]

        zsum = None
        zsq = None
        for i in range(b_tile):
            yparts = []
            for c0 in range(0, c_in, cc):
                xc = x_ref[i, c0:c0 + cc]
                yc = None
                for k in range(ksize):
                    tap = xc if k == pad else _shifted2(xc, k - pad, length)
                    term = tap * ds[k][c0:c0 + cc]
                    yc = term if yc is None else yc + term
                yparts.append(yc.astype(jnp.bfloat16))
            yi = jnp.concatenate(yparts, axis=0)
            z = jnp.dot(pw, yi, preferred_element_type=jnp.float32)
            z_scr[pl.ds(b * b_tile + i, 1)] = z.astype(jnp.bfloat16)[None]
            zsum = z if zsum is None else zsum + z
            q = z * z
            zsq = q if zsq is None else zsq + q
        s1 = jnp.sum(zsum, axis=1, keepdims=True)
        s2 = jnp.sum(zsq, axis=1, keepdims=True)
        st_scr[:, 0:1] = st_scr[:, 0:1] + s1
        st_scr[:, 1:2] = st_scr[:, 1:2] + s2

    @pl.when(p == 1)
    def _apply():
        @pl.when(b == 0)
        def _fold():
            mean = st_scr[:, 0:1] * (1.0 / count)
            var = jnp.maximum(st_scr[:, 1:2] * (1.0 / count) - mean * mean, 0.0)
            inv = jax.lax.rsqrt(var + eps)
            sc_scr[:, 0:1] = inv
            sc_scr[:, 1:2] = -mean * inv

        c_out = sc_scr.shape[0]
        scale = sc_scr[:, 0:1].reshape(1, c_out, 1)
        shift = sc_scr[:, 1:2].reshape(1, c_out, 1)
        z = z_scr[pl.ds(b * b_tile, b_tile)].astype(jnp.float32)
        o_ref[...] = jnp.maximum(z * scale + shift, 0.0)


@functools.partial(jax.jit, static_argnames=("ksize", "eps"))
def _fused(x, dw, pw, *, ksize, eps):
    n, c_in, length = x.shape
    c_out = pw.shape[0]
    l_out = length

    b_tile = 16
    nb = n // b_tile

    kfn = functools.partial(_phased_kernel, ksize=ksize, b_tile=b_tile,
                            count=float(n * l_out), eps=eps)
    out = pl.pallas_call(
        kfn,
        grid=(2, nb),
        out_shape=jax.ShapeDtypeStruct((n, c_out, l_out), jnp.float32),
        in_specs=[
            pl.BlockSpec((b_tile, c_in, length),
                         lambda p, b: (b * jnp.where(p == 0, 1, 0), 0, 0)),
            pl.BlockSpec((c_in, ksize), lambda p, b: (0, 0)),
            pl.BlockSpec((c_out, c_in), lambda p, b: (0, 0)),
        ],
        out_specs=pl.BlockSpec((b_tile, c_out, l_out),
                               lambda p, b: (b * jnp.where(p == 1, 1, 0), 0, 0)),
        scratch_shapes=[
            pltpu.VMEM((n, c_out, l_out), jnp.bfloat16),
            pltpu.VMEM((c_out, 8), jnp.float32),
            pltpu.VMEM((c_out, 8), jnp.float32),
        ],
        compiler_params=pltpu.CompilerParams(
            dimension_semantics=("arbitrary", "arbitrary"),
            vmem_limit_bytes=60 * 1024 * 1024,
            flags={"XLA_TPU_STORE_TO_LOAD_FORWARDING_WINDOW": 12288},
        ),
        cost_estimate=pl.CostEstimate(
            flops=n * l_out * (2 * c_in * ksize + 2 * c_out * c_in + 7 * c_out),
            transcendentals=0,
            bytes_accessed=4 * n * c_in * length + 4 * n * c_out * l_out,
        ),
    )(x, dw, pw)
    return out


def kernel(x, dw, db, pw, pb):
    del db, pb
    n, c_in, length = x.shape
    ksize = dw.reshape(c_in, -1).shape[-1]
    c_out = pw.shape[0]
    x = x.astype(jnp.float32)
    dw = dw.astype(jnp.float32).reshape(c_in, ksize)
    pw = pw.astype(jnp.float32).reshape(c_out, c_in).astype(jnp.bfloat16)
    return _fused(x, dw, pw, ksize=ksize, eps=1e-5)


# speedup vs baseline: 1.2945x; 1.2945x over previous
import functools

import jax
import jax.numpy as jnp
from jax.experimental import pallas as pl
from jax.experimental.pallas import tpu as pltpu


def _shifted2(x, off, length):
    c, _ = x.shape
    if off < 0:
        zc = jnp.zeros((c, -off), x.dtype)
        return jnp.concatenate([zc, x[:, : length + off]], axis=1)
    zc = jnp.zeros((c, off), x.dtype)
    return jnp.concatenate([x[:, off:], zc], axis=1)


def _phased_kernel(x_ref, pwk_ref, o_ref, z_scr, st_scr, sc_scr, *,
                   ksize, b_tile, count, eps):
    p = pl.program_id(0)
    b = pl.program_id(1)

    @pl.when(p == 0)
    def _conv():
        @pl.when(b == 0)
        def _init():
            st_scr[...] = jnp.zeros_like(st_scr)

        c_in, length = x_ref.shape[1], x_ref.shape[2]
        pad = (ksize - 1) // 2
        pwk = [pwk_ref[k] for k in range(ksize)]

        zsum = None
        zsq = None
        zs = []
        for i in range(b_tile):
            xb = x_ref[i].astype(jnp.bfloat16)
            z = None
            for k in range(ksize):
                tap = xb if k == pad else _shifted2(xb, k - pad, length)
                zk = jnp.dot(pwk[k], tap, preferred_element_type=jnp.float32)
                z = zk if z is None else z + zk
            zs.append(z.astype(jnp.bfloat16)[None])
            zsum = z if zsum is None else zsum + z
            q = z * z
            zsq = q if zsq is None else zsq + q
        z_scr[pl.ds(b * b_tile, b_tile)] = jnp.concatenate(zs, axis=0)
        s1 = jnp.sum(zsum, axis=1, keepdims=True)
        s2 = jnp.sum(zsq, axis=1, keepdims=True)
        st_scr[:, 0:1] = st_scr[:, 0:1] + s1
        st_scr[:, 1:2] = st_scr[:, 1:2] + s2

    @pl.when(p == 1)
    def _apply():
        @pl.when(b == 0)
        def _fold():
            mean = st_scr[:, 0:1] * (1.0 / count)
            var = jnp.maximum(st_scr[:, 1:2] * (1.0 / count) - mean * mean, 0.0)
            inv = jax.lax.rsqrt(var + eps)
            sc_scr[:, 0:1] = inv
            sc_scr[:, 1:2] = -mean * inv

        c_out = sc_scr.shape[0]
        scale = sc_scr[:, 0:1].reshape(1, c_out, 1)
        shift = sc_scr[:, 1:2].reshape(1, c_out, 1)
        z = z_scr[pl.ds(b * b_tile, b_tile)].astype(jnp.float32)
        o_ref[...] = jnp.maximum(z * scale + shift, 0.0)


@functools.partial(jax.jit, static_argnames=("ksize", "eps"))
def _fused(x, pwk, *, ksize, eps):
    n, c_in, length = x.shape
    c_out = pwk.shape[1]
    l_out = length

    b_tile = 16
    nb = n // b_tile

    kfn = functools.partial(_phased_kernel, ksize=ksize, b_tile=b_tile,
                            count=float(n * l_out), eps=eps)
    out = pl.pallas_call(
        kfn,
        grid=(2, nb),
        out_shape=jax.ShapeDtypeStruct((n, c_out, l_out), jnp.float32),
        in_specs=[
            pl.BlockSpec((b_tile, c_in, length),
                         lambda p, b: (b * jnp.where(p == 0, 1, 0), 0, 0)),
            pl.BlockSpec((ksize, c_out, c_in), lambda p, b: (0, 0, 0)),
        ],
        out_specs=pl.BlockSpec((b_tile, c_out, l_out),
                               lambda p, b: (b * jnp.where(p == 1, 1, 0), 0, 0)),
        scratch_shapes=[
            pltpu.VMEM((n, c_out, l_out), jnp.bfloat16),
            pltpu.VMEM((c_out, 8), jnp.float32),
            pltpu.VMEM((c_out, 8), jnp.float32),
        ],
        compiler_params=pltpu.CompilerParams(
            dimension_semantics=("arbitrary", "arbitrary"),
            vmem_limit_bytes=60 * 1024 * 1024,
            flags={"XLA_TPU_STORE_TO_LOAD_FORWARDING_WINDOW": 12288},
        ),
        cost_estimate=pl.CostEstimate(
            flops=n * l_out * (2 * ksize * c_out * c_in + 7 * c_out),
            transcendentals=0,
            bytes_accessed=4 * n * c_in * length + 4 * n * c_out * l_out,
        ),
    )(x, pwk)
    return out


def kernel(x, dw, db, pw, pb):
    del db, pb
    n, c_in, length = x.shape
    ksize = dw.reshape(c_in, -1).shape[-1]
    c_out = pw.shape[0]
    x = x.astype(jnp.float32)
    dw = dw.astype(jnp.float32).reshape(c_in, ksize)
    pw = pw.astype(jnp.float32).reshape(c_out, c_in)
    pwk = jnp.stack([pw * dw[:, k][None, :] for k in range(ksize)], axis=0)
    pwk = pwk.astype(jnp.bfloat16)
    return _fused(x, pwk, ksize=ksize, eps=1e-5)
